# baseline (speedup 1.0000x reference)
import functools

import jax
import jax.numpy as jnp
from jax import lax
from jax.experimental import pallas as pl

_I32_MIN = -2147483648
_N_DIMS = 3
_N_LEAVES = 8


def _sortable(f):
    f = jnp.where(f == 0.0, jnp.float32(0.0), f)
    s = lax.bitcast_convert_type(f, jnp.int32)
    return s ^ (lax.shift_right_arithmetic(s, 31) & jnp.int32(0x7FFFFFFF))


_SENT = 2147483647


def _kth(zm, k, nbits, start, R):
    def body(i, vu):
        bit = lax.shift_left(jnp.int32(1), jnp.int32(nbits - 1) - i)
        test_u = vu | bit
        test_s = test_u ^ _I32_MIN
        cnt = jnp.sum((zm < test_s).astype(jnp.int32), axis=1,
                      keepdims=True)
        return jnp.where(cnt >= k, vu, test_u)

    vu0 = jnp.full((R, 1), start, jnp.int32)
    vu = lax.fori_loop(0, nbits, body, vu0)
    return vu ^ _I32_MIN


def _cnt(mask):
    return jnp.sum(mask.astype(jnp.int32), axis=1, keepdims=True)


def _seg_left_mask(keys, segs, k_int, idx_bits, R):
    k = jnp.full((R, 1), k_int, jnp.int32)
    zm = jnp.where(segs, keys[0], jnp.int32(_SENT))
    t = _kth(zm, k, 32, jnp.int32(0), R)
    less = _cnt(zm < t)
    eq = _cnt(zm == t)
    need = k - less
    no_tie = jnp.all(eq == need)

    def fast():
        return (zm <= t).astype(jnp.int32)

    def slow():
        lm = zm < t
        tie = zm == t
        kk = need
        for j, c in enumerate(keys[1:]):
            is_last = j == len(keys) - 2
            nbits, start = (idx_bits, _I32_MIN) if is_last else (32, 0)
            cm = jnp.where(tie, c, jnp.int32(_SENT))
            tj = _kth(cm, kk, nbits, jnp.int32(start), R)
            if is_last:
                lm = lm | (tie & (cm <= tj))
            else:
                lm = lm | (tie & (cm < tj))
                kk = kk - _cnt(cm < tj)
                tie = tie & (cm == tj)
        return lm.astype(jnp.int32)

    return lax.cond(no_tie, fast, slow) != 0


def _body(xt_ref, pos_ref, bias_ref, w1_ref, b1_ref, w2_ref, b2_ref, w3_ref,
          b3_ref, out_ref, *, L, M, FO):
    z = [_sortable(pos_ref[0, a]) for a in range(_N_DIMS)]
    idx = lax.broadcasted_iota(jnp.int32, (L, M), 1)
    ibits = max(M.bit_length() - 1, 1)

    ones = jnp.full((L, M), True)
    left0 = _seg_left_mask([z[0], idx], ones, M // 2, ibits, L)
    bit0 = ~left0

    def dup(a, n):
        return jnp.concatenate([a] * n, axis=0)

    segs1 = jnp.concatenate([left0, bit0], axis=0)
    lm1 = _seg_left_mask([dup(z[1], 2), dup(z[0], 2), dup(idx, 2)], segs1,
                         M // 4, ibits, 2 * L)
    bit1 = (left0 & ~lm1[:L]) | (bit0 & ~lm1[L:])

    seg00 = left0 & ~bit1
    seg01 = left0 & bit1
    seg10 = bit0 & ~bit1
    seg11 = bit0 & bit1
    segs2 = jnp.concatenate([seg00, seg01, seg10, seg11], axis=0)
    lm2 = _seg_left_mask(
        [dup(z[2], 4), dup(z[1], 4), dup(z[0], 4), dup(idx, 4)], segs2,
        M // 8, ibits, 4 * L)
    bit2 = ((seg00 & ~lm2[:L]) | (seg01 & ~lm2[L:2 * L])
            | (seg10 & ~lm2[2 * L:3 * L]) | (seg11 & ~lm2[3 * L:]))

    leaf = (jnp.where(bit0, 4, 0) + jnp.where(bit1, 2, 0)
            + jnp.where(bit2, 1, 0))

    w1 = w1_ref[...]
    w2 = w2_ref[...]
    w3 = w3_ref[...]
    b1v = b1_ref[...]
    b2v = b2_ref[...]
    b3v = b3_ref[...]
    rowmask = (lax.broadcasted_iota(jnp.int32, (FO, 1), 0) % 13) == 0
    neg = jnp.float32(-jnp.inf)
    n_chunks = 8
    cw = M // n_chunks
    for li in range(L):
        xl = xt_ref[0, li]
        raw = lax.dot_general(w1, xl, (((0,), (1,)), ((), ())),
                              preferred_element_type=jnp.float32)
        leaf_row = leaf[li:li + 1, :]
        accs = None
        for c in range(n_chunks):
            hc = raw[:, c * cw:(c + 1) * cw]
            lc = leaf_row[:, c * cw:(c + 1) * cw]
            cols = [
                jnp.max(jnp.where(lc == kk, hc, neg), axis=1, keepdims=True)
                for kk in range(_N_LEAVES)
            ]
            part = jnp.concatenate(cols, axis=1)
            accs = part if accs is None else jnp.maximum(accs, part)
        pooled = jnp.maximum(accs + b1v, 0.0)
        g = lax.dot_general(w2, pooled, (((0,), (0,)), ((), ())),
                            preferred_element_type=jnp.float32)
        g = jnp.maximum(g + b2v, 0.0)
        o = lax.dot_general(w3, g, (((0,), (0,)), ((), ())),
                            preferred_element_type=jnp.float32)
        o = o + b3v
        o = o + jnp.where(rowmask, jnp.abs(bias_ref[0, li, 0]), 0.0)
        out_ref[0, li] = o


def kernel(x, bias_in, division_axis, W1, b1, W2, b2, W3, b3):
    B, L, M, D = x.shape
    H1 = W1.shape[1]
    H2 = W2.shape[1]
    FO = W3.shape[1]
    n_fit = FO // 13

    axes = jnp.mod(division_axis + jnp.arange(_N_DIMS, dtype=jnp.int32),
                   _N_DIMS)
    pos = x[..., 1:1 + _N_DIMS]
    pos_lvl = jnp.take(pos, axes, axis=3)
    pos_mat = pos_lvl.transpose(0, 3, 1, 2)

    body = functools.partial(_body, L=L, M=M, FO=FO)
    out = pl.pallas_call(
        body,
        grid=(B,),
        in_specs=[
            pl.BlockSpec((1, L, M, D), lambda b: (b, 0, 0, 0)),
            pl.BlockSpec((1, _N_DIMS, L, M), lambda b: (b, 0, 0, 0)),
            pl.BlockSpec((1, L, 1), lambda b: (b, 0, 0)),
            pl.BlockSpec((D, H1), lambda b: (0, 0)),
            pl.BlockSpec((H1, 1), lambda b: (0, 0)),
            pl.BlockSpec((H1, H2), lambda b: (0, 0)),
            pl.BlockSpec((H2, 1), lambda b: (0, 0)),
            pl.BlockSpec((H2, FO), lambda b: (0, 0)),
            pl.BlockSpec((FO, 1), lambda b: (0, 0)),
        ],
        out_specs=pl.BlockSpec((1, L, FO, _N_LEAVES), lambda b: (b, 0, 0, 0)),
        out_shape=jax.ShapeDtypeStruct((B, L, FO, _N_LEAVES), jnp.float32),
    )(x, pos_mat, bias_in.reshape(B, L, 1), W1, b1.reshape(H1, 1), W2,
      b2.reshape(H2, 1),
      W3, b3.reshape(FO, 1))

    out = out.transpose(0, 1, 3, 2)
    return out.reshape(B, L, _N_LEAVES, n_fit, 13).reshape(
        B, L, _N_LEAVES * n_fit, 13)


# speedup vs baseline: 1.5614x; 1.5614x over previous
import functools

import jax
import jax.numpy as jnp
from jax import lax
from jax.experimental import pallas as pl

_I32_MIN = -2147483648
_N_DIMS = 3
_N_LEAVES = 8


def _sortable(f):
    f = jnp.where(f == 0.0, jnp.float32(0.0), f)
    s = lax.bitcast_convert_type(f, jnp.int32)
    return s ^ (lax.shift_right_arithmetic(s, 31) & jnp.int32(0x7FFFFFFF))


_SENT = 2147483647


def _kth(zm, k, nbits, start, R):
    def count_below(cand_u):
        return jnp.sum((zm < (cand_u ^ _I32_MIN)).astype(jnp.int32), axis=1,
                       keepdims=True)

    vu = jnp.full((R, 1), start, jnp.int32)
    if nbits % 2:
        c = vu | lax.shift_left(jnp.int32(1), jnp.int32(nbits - 1))
        vu = jnp.where(count_below(c) >= k, vu, c)

    def body(i, vu):
        b1 = lax.shift_left(jnp.int32(1), jnp.int32(nbits - nbits % 2 - 2)
                            - 2 * i)
        c1 = vu | b1
        c2 = vu | (b1 + b1)
        c3 = c2 | b1
        n1 = count_below(c1)
        n2 = count_below(c2)
        n3 = count_below(c3)
        return jnp.where(n3 < k, c3,
                         jnp.where(n2 < k, c2, jnp.where(n1 < k, c1, vu)))

    vu = lax.fori_loop(0, nbits // 2, body, vu)
    return vu ^ _I32_MIN


def _cnt(mask):
    return jnp.sum(mask.astype(jnp.int32), axis=1, keepdims=True)


def _seg_left_mask(keys, segs, k_int, idx_bits, R):
    k = jnp.full((R, 1), k_int, jnp.int32)
    zm = jnp.where(segs, keys[0], jnp.int32(_SENT))
    t = _kth(zm, k, 32, jnp.int32(0), R)
    less = _cnt(zm < t)
    eq = _cnt(zm == t)
    need = k - less
    no_tie = jnp.all(eq == need)

    def fast():
        return (zm <= t).astype(jnp.int32)

    def slow():
        lm = zm < t
        tie = zm == t
        kk = need
        for j, c in enumerate(keys[1:]):
            is_last = j == len(keys) - 2
            nbits, start = (idx_bits, _I32_MIN) if is_last else (32, 0)
            cm = jnp.where(tie, c, jnp.int32(_SENT))
            tj = _kth(cm, kk, nbits, jnp.int32(start), R)
            if is_last:
                lm = lm | (tie & (cm <= tj))
            else:
                lm = lm | (tie & (cm < tj))
                kk = kk - _cnt(cm < tj)
                tie = tie & (cm == tj)
        return lm.astype(jnp.int32)

    return lax.cond(no_tie, fast, slow) != 0


def _body(xt_ref, pos_ref, bias_ref, w1_ref, b1_ref, w2_ref, b2_ref, w3_ref,
          b3_ref, out_ref, *, L, M, FO):
    z = [_sortable(pos_ref[0, a]) for a in range(_N_DIMS)]
    idx = lax.broadcasted_iota(jnp.int32, (L, M), 1)
    ibits = max(M.bit_length() - 1, 1)

    ones = jnp.full((L, M), True)
    left0 = _seg_left_mask([z[0], idx], ones, M // 2, ibits, L)
    bit0 = ~left0

    def dup(a, n):
        return jnp.concatenate([a] * n, axis=0)

    segs1 = jnp.concatenate([left0, bit0], axis=0)
    lm1 = _seg_left_mask([dup(z[1], 2), dup(z[0], 2), dup(idx, 2)], segs1,
                         M // 4, ibits, 2 * L)
    bit1 = (left0 & ~lm1[:L]) | (bit0 & ~lm1[L:])

    seg00 = left0 & ~bit1
    seg01 = left0 & bit1
    seg10 = bit0 & ~bit1
    seg11 = bit0 & bit1
    segs2 = jnp.concatenate([seg00, seg01, seg10, seg11], axis=0)
    lm2 = _seg_left_mask(
        [dup(z[2], 4), dup(z[1], 4), dup(z[0], 4), dup(idx, 4)], segs2,
        M // 8, ibits, 4 * L)
    bit2 = ((seg00 & ~lm2[:L]) | (seg01 & ~lm2[L:2 * L])
            | (seg10 & ~lm2[2 * L:3 * L]) | (seg11 & ~lm2[3 * L:]))

    leaf = (jnp.where(bit0, 4, 0) + jnp.where(bit1, 2, 0)
            + jnp.where(bit2, 1, 0))

    w1 = w1_ref[...]
    w2 = w2_ref[...]
    w3 = w3_ref[...]
    b1v = b1_ref[...]
    b2v = b2_ref[...]
    b3v = b3_ref[...]
    rowmask = (lax.broadcasted_iota(jnp.int32, (FO, 1), 0) % 13) == 0
    neg = jnp.float32(-jnp.inf)
    n_chunks = 8
    cw = M // n_chunks
    for li in range(L):
        xlt = xt_ref[0, li]
        raw = lax.dot_general(w1, xlt, (((0,), (0,)), ((), ())),
                              preferred_element_type=jnp.float32)
        leaf_row = leaf[li:li + 1, :]
        accs = None
        for c in range(n_chunks):
            hc = raw[:, c * cw:(c + 1) * cw]
            lc = leaf_row[:, c * cw:(c + 1) * cw]
            cols = [
                jnp.max(jnp.where(lc == kk, hc, neg), axis=1, keepdims=True)
                for kk in range(_N_LEAVES)
            ]
            part = jnp.concatenate(cols, axis=1)
            accs = part if accs is None else jnp.maximum(accs, part)
        pooled = jnp.maximum(accs + b1v, 0.0)
        g = lax.dot_general(w2, pooled, (((0,), (0,)), ((), ())),
                            preferred_element_type=jnp.float32)
        g = jnp.maximum(g + b2v, 0.0)
        o = lax.dot_general(w3, g, (((0,), (0,)), ((), ())),
                            preferred_element_type=jnp.float32)
        o = o + b3v
        o = o + jnp.where(rowmask, jnp.abs(bias_ref[0, li, 0]), 0.0)
        out_ref[0, li] = o


def kernel(x, bias_in, division_axis, W1, b1, W2, b2, W3, b3):
    B, L, M, D = x.shape
    H1 = W1.shape[1]
    H2 = W2.shape[1]
    FO = W3.shape[1]
    n_fit = FO // 13

    axes = jnp.mod(division_axis + jnp.arange(_N_DIMS, dtype=jnp.int32),
                   _N_DIMS)
    pos = x[..., 1:1 + _N_DIMS]
    pos_lvl = jnp.take(pos, axes, axis=3)
    pos_mat = pos_lvl.transpose(0, 3, 1, 2)
    xt = x.transpose(0, 1, 3, 2)

    body = functools.partial(_body, L=L, M=M, FO=FO)
    out = pl.pallas_call(
        body,
        grid=(B,),
        in_specs=[
            pl.BlockSpec((1, L, D, M), lambda b: (b, 0, 0, 0)),
            pl.BlockSpec((1, _N_DIMS, L, M), lambda b: (b, 0, 0, 0)),
            pl.BlockSpec((1, L, 1), lambda b: (b, 0, 0)),
            pl.BlockSpec((D, H1), lambda b: (0, 0)),
            pl.BlockSpec((H1, 1), lambda b: (0, 0)),
            pl.BlockSpec((H1, H2), lambda b: (0, 0)),
            pl.BlockSpec((H2, 1), lambda b: (0, 0)),
            pl.BlockSpec((H2, FO), lambda b: (0, 0)),
            pl.BlockSpec((FO, 1), lambda b: (0, 0)),
        ],
        out_specs=pl.BlockSpec((1, L, FO, _N_LEAVES), lambda b: (b, 0, 0, 0)),
        out_shape=jax.ShapeDtypeStruct((B, L, FO, _N_LEAVES), jnp.float32),
    )(xt, pos_mat, bias_in.reshape(B, L, 1), W1, b1.reshape(H1, 1), W2,
      b2.reshape(H2, 1),
      W3, b3.reshape(FO, 1))

    out = out.transpose(0, 1, 3, 2)
    return out.reshape(B, L, _N_LEAVES, n_fit, 13).reshape(
        B, L, _N_LEAVES * n_fit, 13)
